# baseline (speedup 1.0000x reference)
import functools

import jax
import jax.numpy as jnp
from jax import lax
from jax.experimental import pallas as pl
from jax.experimental.pallas import tpu as pltpu
from jax.experimental.pallas import tpu_sc as plsc

NC, NS, L = 2, 16, 16
NW = NC * NS
B, S, D = 1024, 200, 50
DP = 64
TOKENS = B * S
PER_W = TOKENS // NW
CHUNK = S
N_CHUNKS = PER_W // CHUNK
CWORDS = CHUNK * D


def _sc_embed(ids_flat, wt_pad, pos_flat):
    mesh = plsc.VectorSubcoreMesh(core_axis_name="c", subcore_axis_name="s")

    @functools.partial(
        pl.kernel,
        out_type=jax.ShapeDtypeStruct((TOKENS * D,), jnp.float32),
        mesh=mesh,
        scratch_types=[
            pltpu.VMEM((CHUNK,), jnp.int32),
            pltpu.VMEM((CHUNK, DP), jnp.float32),
            pltpu.VMEM((CWORDS,), jnp.float32),
            pltpu.VMEM((CWORDS,), jnp.float32),
            pltpu.SemaphoreType.DMA,
        ],
        compiler_params=pltpu.CompilerParams(use_tc_tiling_on_sc=False),
    )
    def k(ids_hbm, tab_hbm, pos_hbm, out_hbm, idx_v, rows_v, o_v, pos_v, sem):
        wid = lax.axis_index("s") * NC + lax.axis_index("c")
        pltpu.sync_copy(pos_hbm, pos_v)
        base_w = wid * PER_W

        @pl.loop(0, N_CHUNKS)
        def _(j):
            base = base_w + j * CHUNK
            pltpu.sync_copy(ids_hbm.at[pl.ds(base, CHUNK)], idx_v)
            g0 = pltpu.async_copy(
                tab_hbm.at[idx_v.at[pl.ds(0, 128)]],
                rows_v.at[pl.ds(0, 128)], sem)
            g1 = pltpu.async_copy(
                tab_hbm.at[idx_v.at[pl.ds(128, 72)]],
                rows_v.at[pl.ds(128, 72)], sem)
            g0.wait()
            g1.wait()

            @pl.loop(0, CHUNK)
            def _(r):
                for c in (0, 16, 32, 34):
                    o_v.at[pl.ds(r * D + c, L)][...] = (
                        rows_v.at[(pl.ds(r, 1), pl.ds(c, L))][...].reshape(L)
                        + pos_v.at[pl.ds(r * D + c, L)][...])

            pltpu.sync_copy(o_v, out_hbm.at[pl.ds(base * D, CWORDS)])

    return k(ids_flat, wt_pad, pos_flat)


def kernel(token_ids, word_table, pos_table):
    ids_flat = token_ids.reshape(-1).astype(jnp.int32)
    wt_pad = jnp.pad(word_table, ((0, 0), (0, DP - D)))
    pos_flat = pos_table[:S].reshape(-1)
    out = _sc_embed(ids_flat, wt_pad, pos_flat)
    return out.reshape(B, S, D)


# speedup vs baseline: 1.1926x; 1.1926x over previous
import functools

import jax
import jax.numpy as jnp
from jax import lax
from jax.experimental import pallas as pl
from jax.experimental.pallas import tpu as pltpu
from jax.experimental.pallas import tpu_sc as plsc

NC, NS, L = 2, 16, 16
NW = NC * NS
B, S, D = 1024, 200, 50
DP = 64
TOKENS = B * S
PER_W = TOKENS // NW
CHUNK = S
N_CHUNKS = PER_W // CHUNK
G0, G1 = 128, CHUNK - 128


def _sc_embed(ids_flat, wt_pad, pos_rows):
    mesh = plsc.VectorSubcoreMesh(core_axis_name="c", subcore_axis_name="s")

    @functools.partial(
        pl.kernel,
        out_type=jax.ShapeDtypeStruct((B, S, D), jnp.float32),
        mesh=mesh,
        scratch_types=[
            pltpu.VMEM((PER_W,), jnp.int32),
            pltpu.VMEM((CHUNK, DP), jnp.float32),
            pltpu.VMEM((CHUNK, DP), jnp.float32),
            pltpu.VMEM((CHUNK, D), jnp.float32),
            pltpu.VMEM((CHUNK, D), jnp.float32),
            pltpu.VMEM((S, D), jnp.float32),
            pltpu.SemaphoreType.DMA,
            pltpu.SemaphoreType.DMA,
            pltpu.SemaphoreType.DMA,
            pltpu.SemaphoreType.DMA,
        ],
        compiler_params=pltpu.CompilerParams(use_tc_tiling_on_sc=False),
    )
    def k(ids_hbm, tab_hbm, pos_hbm, out_hbm,
          ids_v, rows0, rows1, out0, out1, pos_v, gs0, gs1, os0, os1):
        wid = lax.axis_index("s") * NC + lax.axis_index("c")
        base_w = wid * PER_W
        seq_w = wid * N_CHUNKS
        pltpu.sync_copy(pos_hbm, pos_v)
        pltpu.sync_copy(ids_hbm.at[pl.ds(base_w, PER_W)], ids_v)

        def _gather_descs(j, rows_b, sem):
            c0 = pltpu.make_async_copy(
                tab_hbm.at[ids_v.at[pl.ds(j * CHUNK, G0)]],
                rows_b.at[pl.ds(0, G0)], sem)
            c1 = pltpu.make_async_copy(
                tab_hbm.at[ids_v.at[pl.ds(j * CHUNK + G0, G1)]],
                rows_b.at[pl.ds(G0, G1)], sem)
            return c0, c1

        def gather(j, rows_b, sem):
            for c in _gather_descs(j, rows_b, sem):
                c.start()

        def wait_gather(j, rows_b, sem):
            for c in _gather_descs(j, rows_b, sem):
                c.wait()

        def compute(rows_b, out_b):
            @pl.loop(0, CHUNK)
            def _(r):
                for c in (0, 16, 32, 34):
                    slc = (pl.ds(r, 1), pl.ds(c, L))
                    out_b.at[slc][...] = (
                        rows_b.at[slc][...] + pos_v.at[slc][...])

        def put(j, out_b, sem):
            pltpu.make_async_copy(out_b, out_hbm.at[seq_w + j], sem).start()

        def wait_put(j, out_b, sem):
            pltpu.make_async_copy(out_b, out_hbm.at[seq_w + j], sem).wait()

        gather(0, rows0, gs0)

        @pl.loop(0, N_CHUNKS // 2)
        def _(kk):
            j0 = 2 * kk
            j1 = j0 + 1

            wait_gather(j0, rows0, gs0)
            gather(j1, rows1, gs1)

            @pl.when(kk > 0)
            def _():
                wait_put(j0 - 2, out0, os0)
            compute(rows0, out0)
            put(j0, out0, os0)

            wait_gather(j1, rows1, gs1)

            @pl.when(j1 < N_CHUNKS - 1)
            def _():
                gather(j1 + 1, rows0, gs0)

            @pl.when(kk > 0)
            def _():
                wait_put(j1 - 2, out1, os1)
            compute(rows1, out1)
            put(j1, out1, os1)

        wait_put(N_CHUNKS - 2, out0, os0)
        wait_put(N_CHUNKS - 1, out1, os1)

    return k(ids_flat, wt_pad, pos_rows)


def kernel(token_ids, word_table, pos_table):
    ids_flat = token_ids.reshape(-1).astype(jnp.int32)
    wt_pad = jnp.pad(word_table, ((0, 0), (0, DP - D)))
    return _sc_embed(ids_flat, wt_pad, pos_table[:S])
